# baseline (speedup 1.0000x reference)
import functools

import jax
import jax.numpy as jnp
from jax import lax
from jax.experimental import pallas as pl
from jax.experimental.pallas import tpu as pltpu
from jax.experimental.pallas import tpu_sc as plsc

_N_FIELDS = 26
_VOCAB = 100000
_DIM = 64
_BATCH = 16384
_NC = 2
_NS = 16
_NW = _NC * _NS
_BPW = _BATCH // _NW
_LANES = 16


def _sc_body(xt_hbm, tab_hbm, out_hbm, idx_v, idx_a, idx0, idx1,
             rows0, rows1, acc, sem_a, sem0, sem1):
    c = lax.axis_index("c")
    s = lax.axis_index("s")
    wid = s * _NC + c
    base = wid * _BPW

    pltpu.sync_copy(xt_hbm.at[:, pl.ds(base, _BPW)], idx_v)

    def fill_idx(f, dst):
        def body(i, _):
            sl = pl.ds(i * _LANES, _LANES)
            dst[sl] = idx_v[f, sl] + f * _VOCAB
            return 0
        lax.fori_loop(0, _BPW // _LANES, body, 0)

    def gather(idx_ref, dst, sem):
        return pltpu.async_copy(tab_hbm.at[idx_ref], dst, sem)

    bufs = (rows0, rows1)
    idxs = (idx0, idx1)
    sems = (sem0, sem1)

    fill_idx(0, idx_a)
    cp_acc = gather(idx_a, acc, sem_a)
    fill_idx(1, idx0)
    fill_idx(2, idx1)
    cp = [gather(idx0, rows0, sem0), gather(idx1, rows1, sem1)]
    cp_acc.wait()

    for f in range(1, _N_FIELDS):
        k = (f - 1) % 2
        cp[k].wait()
        buf = bufs[k]

        def acc_body(i, _, buf=buf):
            for j in range(_DIM // _LANES):
                sl = pl.ds(j * _LANES, _LANES)
                plsc.addupdate(acc.at[i, sl], buf[i, sl])
            return 0
        lax.fori_loop(0, _BPW, acc_body, 0)

        if f + 2 < _N_FIELDS:
            fill_idx(f + 2, idxs[k])
            cp[k] = gather(idxs[k], bufs[k], sems[k])

    pltpu.sync_copy(acc, out_hbm.at[pl.ds(base, _BPW)])


@jax.jit
def kernel(x, tables):
    xt = x.T
    tab = tables.reshape(_N_FIELDS * _VOCAB, _DIM)
    run = functools.partial(
        pl.kernel,
        out_type=jax.ShapeDtypeStruct((_BATCH, _DIM), jnp.float32),
        mesh=plsc.VectorSubcoreMesh(core_axis_name="c", subcore_axis_name="s"),
        compiler_params=pltpu.CompilerParams(use_tc_tiling_on_sc=False),
        scratch_types=[
            pltpu.VMEM((_N_FIELDS, _BPW), jnp.int32),
            pltpu.VMEM((_BPW,), jnp.int32),
            pltpu.VMEM((_BPW,), jnp.int32),
            pltpu.VMEM((_BPW,), jnp.int32),
            pltpu.VMEM((_BPW, _DIM), jnp.float32),
            pltpu.VMEM((_BPW, _DIM), jnp.float32),
            pltpu.VMEM((_BPW, _DIM), jnp.float32),
            pltpu.SemaphoreType.DMA,
            pltpu.SemaphoreType.DMA,
            pltpu.SemaphoreType.DMA,
        ],
    )(_sc_body)
    return run(xt, tab)


# speedup vs baseline: 1.0166x; 1.0166x over previous
import functools

import jax
import jax.numpy as jnp
from jax import lax
from jax.experimental import pallas as pl
from jax.experimental.pallas import tpu as pltpu
from jax.experimental.pallas import tpu_sc as plsc

_N_FIELDS = 26
_VOCAB = 100000
_DIM = 64
_BATCH = 16384
_NC = 2
_NS = 16
_NW = _NC * _NS
_BPW = _BATCH // _NW
_LANES = 16


def _sc_body(xt_hbm, tab_hbm, out_hbm, *rest):
    idx_v = rest[0]
    idx_f = rest[1:1 + _N_FIELDS]
    acc = rest[1 + _N_FIELDS]
    sem = rest[2 + _N_FIELDS]

    c = lax.axis_index("c")
    s = lax.axis_index("s")
    wid = s * _NC + c
    base = wid * _BPW

    pltpu.sync_copy(xt_hbm.at[:, pl.ds(base, _BPW)], idx_v)

    zeros = jnp.zeros((_LANES,), jnp.float32)

    def zero_body(i, _):
        for j in range(_DIM // _LANES):
            acc[i, pl.ds(j * _LANES, _LANES)] = zeros
        return 0
    lax.fori_loop(0, _BPW, zero_body, 0)

    copies = []
    for f in range(_N_FIELDS):
        dst = idx_f[f]

        def fill_body(i, _, f=f, dst=dst):
            sl = pl.ds(i * _LANES, _LANES)
            dst[sl] = idx_v[f, sl] + f * _VOCAB
            return 0
        lax.fori_loop(0, _BPW // _LANES, fill_body, 0)
        copies.append(pltpu.async_copy(tab_hbm.at[dst], acc, sem, add=True))

    for cp in copies:
        cp.wait()

    pltpu.sync_copy(acc, out_hbm.at[pl.ds(base, _BPW)])


@jax.jit
def kernel(x, tables):
    xt = x.T
    tab = tables.reshape(_N_FIELDS * _VOCAB, _DIM)
    scratch = [pltpu.VMEM((_N_FIELDS, _BPW), jnp.int32)]
    scratch += [pltpu.VMEM((_BPW,), jnp.int32) for _ in range(_N_FIELDS)]
    scratch += [pltpu.VMEM((_BPW, _DIM), jnp.float32), pltpu.SemaphoreType.DMA]
    run = functools.partial(
        pl.kernel,
        out_type=jax.ShapeDtypeStruct((_BATCH, _DIM), jnp.float32),
        mesh=plsc.VectorSubcoreMesh(core_axis_name="c", subcore_axis_name="s"),
        compiler_params=pltpu.CompilerParams(use_tc_tiling_on_sc=False),
        scratch_types=scratch,
    )(_sc_body)
    return run(xt, tab)
